# baseline (speedup 1.0000x reference)
import functools

import jax
import jax.numpy as jnp
from jax import lax
from jax.experimental import pallas as pl
from jax.experimental.pallas import tpu as pltpu
from jax.experimental.pallas import tpu_sc as plsc

B, C, L = 32, 256, 576
K = 8192
N = B * L
KC = 1024
NKC = K // KC
NW = 32
ROWS_PER_W = N // NW
CH = 96
NCH = ROWS_PER_W // CH
BETA = 0.25


def _argmin_body(x_ref, emb_ref, idx_ref, loss_ref, e2_ref, ee_ref):
    b = pl.program_id(0)

    @pl.when(b == 0)
    def _():
        e = emb_ref[...]
        e2_ref[...] = e + e
        ee_ref[...] = jnp.sum(e * e, axis=1)[None, :]

    xb = x_ref[0]
    xx = jnp.sum(xb * xb, axis=0)
    xxb = jnp.broadcast_to(xx[:, None], (L, KC))

    rmin = None
    rkc = None
    for kc in range(NKC):
        e2 = e2_ref[pl.ds(kc * KC, KC), :]
        ee = ee_ref[0, pl.ds(kc * KC, KC)]
        dots2 = lax.dot_general(
            xb, e2, (((0,), (1,)), ((), ())),
            preferred_element_type=jnp.float32,
        )
        d2 = (xxb + ee[None, :]) - dots2
        if rmin is None:
            rmin, rkc = d2, jnp.zeros((L, KC), jnp.int32)
        else:
            upd = d2 < rmin
            rmin = jnp.minimum(d2, rmin)
            rkc = jnp.where(upd, kc, rkc)

    minval = jnp.min(rmin, axis=1)
    j = lax.broadcasted_iota(jnp.int32, (L, KC), 1)
    gk = rkc * KC + j
    cand = jnp.where(rmin == minval[:, None], gk, K)
    idx_ref[0, 0, :] = jnp.min(cand, axis=1)

    @pl.when(b == 0)
    def _():
        loss_ref[0, 0] = 0.0

    loss_ref[0, 0] += jnp.sum(jnp.maximum(minval, 0.0))


def _argmin_call(x, emb_w):
    nb = x.shape[0]
    return pl.pallas_call(
        _argmin_body,
        grid=(nb,),
        in_specs=[
            pl.BlockSpec((1, C, L), lambda b: (b, 0, 0)),
            pl.BlockSpec((K, C), lambda b: (0, 0)),
        ],
        out_specs=[
            pl.BlockSpec((1, 1, L), lambda b: (b, 0, 0)),
            pl.BlockSpec(block_shape=(1, 1), index_map=lambda b: (0, 0),
                         memory_space=pltpu.SMEM),
        ],
        out_shape=[
            jax.ShapeDtypeStruct((nb, 1, L), jnp.int32),
            jax.ShapeDtypeStruct((1, 1), jnp.float32),
        ],
        scratch_shapes=[
            pltpu.VMEM((K, C), jnp.float32),
            pltpu.VMEM((1, K), jnp.float32),
        ],
        compiler_params=pltpu.CompilerParams(
            dimension_semantics=("arbitrary",),
        ),
    )(x, emb_w)


def _gather_body(nrows, emb_hbm, idx_hbm, out_hbm,
                 idx_v0, idx_v1, rows_v0, rows_v1, sem0, sem1):
    rpw = nrows // NW
    nch = rpw // CH
    wid = lax.axis_index("s") * 2 + lax.axis_index("c")
    base = wid * rpw
    idx_v = (idx_v0, idx_v1)
    rows_v = (rows_v0, rows_v1)
    sem = (sem0, sem1)
    pltpu.sync_copy(idx_hbm.at[pl.ds(base, CH)], idx_v0)
    cp = pltpu.async_copy(emb_hbm.at[idx_v0], rows_v0, sem0)
    for ch in range(1, nch):
        p = ch % 2
        off = base + ch * CH
        pltpu.sync_copy(idx_hbm.at[pl.ds(off, CH)], idx_v[p])
        nxt = pltpu.async_copy(emb_hbm.at[idx_v[p]], rows_v[p], sem[p])
        cp.wait()
        pltpu.sync_copy(rows_v[1 - p], out_hbm.at[pl.ds(off - CH, CH)])
        cp = nxt
    cp.wait()
    pltpu.sync_copy(rows_v[(nch - 1) % 2],
                    out_hbm.at[pl.ds(base + (nch - 1) * CH, CH)])


def _gather_call(emb_w, idx_flat):
    nrows = idx_flat.shape[0]
    call = functools.partial(
        pl.kernel,
        out_type=jax.ShapeDtypeStruct((nrows, C), jnp.float32),
        mesh=plsc.VectorSubcoreMesh(core_axis_name="c", subcore_axis_name="s",
                                    num_cores=2, num_subcores=16),
        scratch_types=[
            pltpu.VMEM((CH,), jnp.int32),
            pltpu.VMEM((CH,), jnp.int32),
            pltpu.VMEM((CH, C), jnp.float32),
            pltpu.VMEM((CH, C), jnp.float32),
            pltpu.SemaphoreType.DMA,
            pltpu.SemaphoreType.DMA,
        ],
    )(functools.partial(_gather_body, nrows))
    return call(emb_w, idx_flat)


def kernel(x, emb_w):
    h = B // 2
    idxA3, sA = _argmin_call(x[:h], emb_w)
    gA = _gather_call(emb_w, idxA3.reshape(h * L))
    idxB3, sB = _argmin_call(x[h:], emb_w)
    gB = _gather_call(emb_w, idxB3.reshape(h * L))
    outA = jnp.transpose(gA.reshape(h, L, C), (0, 2, 1))
    outB = jnp.transpose(gB.reshape(h, L, C), (0, 2, 1))
    x_q_out = jnp.concatenate([outA, outB], axis=0)
    idxs = jnp.concatenate([idxA3.reshape(h, L), idxB3.reshape(h, L)], axis=0)
    s = sA.reshape(()) + sB.reshape(())
    q = s / jnp.float32(N * C)
    return (x_q_out, idxs, q + BETA * q)


# speedup vs baseline: 1.0111x; 1.0111x over previous
import functools

import jax
import jax.numpy as jnp
from jax import lax
from jax.experimental import pallas as pl
from jax.experimental.pallas import tpu as pltpu
from jax.experimental.pallas import tpu_sc as plsc

B, C, L = 32, 256, 576
K = 8192
N = B * L
KC = 2048
NKC = K // KC
NW = 32
ROWS_PER_W = N // NW
CH = 96
NCH = ROWS_PER_W // CH
BETA = 0.25


def _argmin_body(x_ref, emb_ref, idx_ref, loss_ref, e2_ref, ee_ref):
    b = pl.program_id(0)

    @pl.when(b == 0)
    def _():
        e = emb_ref[...]
        e2_ref[...] = e + e
        ee_ref[...] = jnp.sum(e * e, axis=1)[None, :]

    xb = x_ref[0]
    xx = jnp.sum(xb * xb, axis=0)
    xxb = jnp.broadcast_to(xx[:, None], (L, KC))

    rmin = None
    rkc = None
    for kc in range(NKC):
        e2 = e2_ref[pl.ds(kc * KC, KC), :]
        ee = ee_ref[0, pl.ds(kc * KC, KC)]
        dots2 = lax.dot_general(
            xb, e2, (((0,), (1,)), ((), ())),
            preferred_element_type=jnp.float32,
        )
        d2 = (xxb + ee[None, :]) - dots2
        if rmin is None:
            rmin, rkc = d2, jnp.zeros((L, KC), jnp.int32)
        else:
            upd = d2 < rmin
            rmin = jnp.minimum(d2, rmin)
            rkc = jnp.where(upd, kc, rkc)

    minval = jnp.min(rmin, axis=1)
    j = lax.broadcasted_iota(jnp.int32, (L, KC), 1)
    gk = rkc * KC + j
    cand = jnp.where(rmin == minval[:, None], gk, K)
    idx_ref[0, 0, :] = jnp.min(cand, axis=1)

    @pl.when(b == 0)
    def _():
        loss_ref[0, 0] = 0.0

    loss_ref[0, 0] += jnp.sum(jnp.maximum(minval, 0.0))

    @pl.when(b == B - 1)
    def _():
        q = loss_ref[0, 0] / jnp.float32(N * C)
        loss_ref[0, 0] = q + BETA * q


def _argmin_call(x, emb_w):
    return pl.pallas_call(
        _argmin_body,
        grid=(B,),
        in_specs=[
            pl.BlockSpec((1, C, L), lambda b: (b, 0, 0)),
            pl.BlockSpec((K, C), lambda b: (0, 0)),
        ],
        out_specs=[
            pl.BlockSpec((1, 1, L), lambda b: (b, 0, 0)),
            pl.BlockSpec(block_shape=(1, 1), index_map=lambda b: (0, 0),
                         memory_space=pltpu.SMEM),
        ],
        out_shape=[
            jax.ShapeDtypeStruct((B, 1, L), jnp.int32),
            jax.ShapeDtypeStruct((1, 1), jnp.float32),
        ],
        scratch_shapes=[
            pltpu.VMEM((K, C), jnp.float32),
            pltpu.VMEM((1, K), jnp.float32),
        ],
        compiler_params=pltpu.CompilerParams(
            dimension_semantics=("arbitrary",),
        ),
    )(x, emb_w)


def _gather_body(emb_hbm, idx_hbm, out_hbm,
                 idx_v0, idx_v1, rows_v0, rows_v1, sem0, sem1):
    wid = lax.axis_index("s") * 2 + lax.axis_index("c")
    base = wid * ROWS_PER_W
    idx_v = (idx_v0, idx_v1)
    rows_v = (rows_v0, rows_v1)
    sem = (sem0, sem1)
    pltpu.sync_copy(idx_hbm.at[pl.ds(base, CH)], idx_v0)
    cp = pltpu.async_copy(emb_hbm.at[idx_v0], rows_v0, sem0)
    for ch in range(1, NCH):
        p = ch % 2
        off = base + ch * CH
        pltpu.sync_copy(idx_hbm.at[pl.ds(off, CH)], idx_v[p])
        nxt = pltpu.async_copy(emb_hbm.at[idx_v[p]], rows_v[p], sem[p])
        cp.wait()
        pltpu.sync_copy(rows_v[1 - p], out_hbm.at[pl.ds(off - CH, CH)])
        cp = nxt
    cp.wait()
    pltpu.sync_copy(rows_v[(NCH - 1) % 2],
                    out_hbm.at[pl.ds(base + (NCH - 1) * CH, CH)])


def _gather_call(emb_w, idx_flat):
    call = functools.partial(
        pl.kernel,
        out_type=jax.ShapeDtypeStruct((N, C), jnp.float32),
        mesh=plsc.VectorSubcoreMesh(core_axis_name="c", subcore_axis_name="s",
                                    num_cores=2, num_subcores=16),
        scratch_types=[
            pltpu.VMEM((CH,), jnp.int32),
            pltpu.VMEM((CH,), jnp.int32),
            pltpu.VMEM((CH, C), jnp.float32),
            pltpu.VMEM((CH, C), jnp.float32),
            pltpu.SemaphoreType.DMA,
            pltpu.SemaphoreType.DMA,
        ],
    )(_gather_body)
    return call(emb_w, idx_flat)


def kernel(x, emb_w):
    idx3, loss = _argmin_call(x, emb_w)
    idxs = idx3.reshape(B, L)
    x_q = _gather_call(emb_w, idxs.reshape(N))
    x_q_out = jnp.transpose(x_q.reshape(B, L, C), (0, 2, 1))
    return (x_q_out, idxs, loss.reshape(()))


# speedup vs baseline: 1.1440x; 1.1315x over previous
import functools

import jax
import jax.numpy as jnp
from jax import lax
from jax.experimental import pallas as pl
from jax.experimental.pallas import tpu as pltpu
from jax.experimental.pallas import tpu_sc as plsc

B, C, L = 32, 256, 576
K = 8192
N = B * L
KC = 512
NKC = K // KC
NW = 32
ROWS_PER_W = N // NW
CH = 96
NCH = ROWS_PER_W // CH
BETA = 0.25


def _argmin_body(x_ref, emb_ref, idx_ref, loss_ref, e2_ref, ee_ref):
    b = pl.program_id(0)

    @pl.when(b == 0)
    def _():
        e = emb_ref[...]
        e2_ref[...] = e + e
        ee_ref[...] = jnp.sum(e * e, axis=1)[None, :]

    xb = x_ref[0]
    xx = jnp.sum(xb * xb, axis=0)
    xxb = jnp.broadcast_to(xx[:, None], (L, KC))

    rmin = None
    rkc = None
    for kc in range(NKC):
        e2 = e2_ref[pl.ds(kc * KC, KC), :]
        ee = ee_ref[0, pl.ds(kc * KC, KC)]
        dots2 = lax.dot_general(
            xb, e2, (((0,), (1,)), ((), ())),
            preferred_element_type=jnp.float32,
        )
        d2 = (xxb + ee[None, :]) - dots2
        if rmin is None:
            rmin, rkc = d2, jnp.zeros((L, KC), jnp.int32)
        else:
            upd = d2 < rmin
            rmin = jnp.minimum(d2, rmin)
            rkc = jnp.where(upd, kc, rkc)

    minval = jnp.min(rmin, axis=1)
    j = lax.broadcasted_iota(jnp.int32, (L, KC), 1)
    gk = rkc * KC + j
    cand = jnp.where(rmin == minval[:, None], gk, K)
    idx_ref[0, 0, :] = jnp.min(cand, axis=1)

    @pl.when(b == 0)
    def _():
        loss_ref[0, 0] = 0.0

    loss_ref[0, 0] += jnp.sum(jnp.maximum(minval, 0.0))

    @pl.when(b == B - 1)
    def _():
        q = loss_ref[0, 0] / jnp.float32(N * C)
        loss_ref[0, 0] = q + BETA * q


def _argmin_call(x, emb_w):
    return pl.pallas_call(
        _argmin_body,
        grid=(B,),
        in_specs=[
            pl.BlockSpec((1, C, L), lambda b: (b, 0, 0)),
            pl.BlockSpec((K, C), lambda b: (0, 0)),
        ],
        out_specs=[
            pl.BlockSpec((1, 1, L), lambda b: (b, 0, 0)),
            pl.BlockSpec(block_shape=(1, 1), index_map=lambda b: (0, 0),
                         memory_space=pltpu.SMEM),
        ],
        out_shape=[
            jax.ShapeDtypeStruct((B, 1, L), jnp.int32),
            jax.ShapeDtypeStruct((1, 1), jnp.float32),
        ],
        scratch_shapes=[
            pltpu.VMEM((K, C), jnp.float32),
            pltpu.VMEM((1, K), jnp.float32),
        ],
        compiler_params=pltpu.CompilerParams(
            dimension_semantics=("arbitrary",),
        ),
    )(x, emb_w)


def _gather_body(emb_hbm, idx_hbm, out_hbm,
                 idx_v0, idx_v1, rows_v0, rows_v1, sem0, sem1):
    wid = lax.axis_index("s") * 2 + lax.axis_index("c")
    base = wid * ROWS_PER_W
    idx_v = (idx_v0, idx_v1)
    rows_v = (rows_v0, rows_v1)
    sem = (sem0, sem1)
    pltpu.sync_copy(idx_hbm.at[pl.ds(base, CH)], idx_v0)
    cp = pltpu.async_copy(emb_hbm.at[idx_v0], rows_v0, sem0)
    for ch in range(1, NCH):
        p = ch % 2
        off = base + ch * CH
        pltpu.sync_copy(idx_hbm.at[pl.ds(off, CH)], idx_v[p])
        nxt = pltpu.async_copy(emb_hbm.at[idx_v[p]], rows_v[p], sem[p])
        cp.wait()
        pltpu.sync_copy(rows_v[1 - p], out_hbm.at[pl.ds(off - CH, CH)])
        cp = nxt
    cp.wait()
    pltpu.sync_copy(rows_v[(NCH - 1) % 2],
                    out_hbm.at[pl.ds(base + (NCH - 1) * CH, CH)])


def _gather_call(emb_w, idx_flat):
    call = functools.partial(
        pl.kernel,
        out_type=jax.ShapeDtypeStruct((N, C), jnp.float32),
        mesh=plsc.VectorSubcoreMesh(core_axis_name="c", subcore_axis_name="s",
                                    num_cores=2, num_subcores=16),
        scratch_types=[
            pltpu.VMEM((CH,), jnp.int32),
            pltpu.VMEM((CH,), jnp.int32),
            pltpu.VMEM((CH, C), jnp.float32),
            pltpu.VMEM((CH, C), jnp.float32),
            pltpu.SemaphoreType.DMA,
            pltpu.SemaphoreType.DMA,
        ],
    )(_gather_body)
    return call(emb_w, idx_flat)


def kernel(x, emb_w):
    idx3, loss = _argmin_call(x, emb_w)
    idxs = idx3.reshape(B, L)
    x_q = _gather_call(emb_w, idxs.reshape(N))
    x_q_out = jnp.transpose(x_q.reshape(B, L, C), (0, 2, 1))
    return (x_q_out, idxs, loss.reshape(()))
